# baseline (speedup 1.0000x reference)
import jax
import jax.numpy as jnp
import numpy as np
from jax.experimental import pallas as pl
from jax.experimental.pallas import tpu as pltpu

_N = 100000
_BL = 50048
_GRID = -(-_N // _BL)

_T = np.arange(0, 336, 1, dtype=np.float32).reshape(7, 6, 8)
_CADD_T = np.zeros((6, 8, 128), np.float32)
_CADD_T[:, :, 0:7] = _T[[0, 2, 1, 3, 4, 5, 6]].transpose(1, 2, 0)


def _body(x_ref, c_ref, o_ref):
    o_ref[...] = x_ref[...]
    @pl.when(pl.program_id(0) == 0)
    def _():
        o_ref[:, :, 0:128] = o_ref[:, :, 0:128] + c_ref[...]


def kernel(x):
    caddT = jnp.asarray(_CADD_T)
    xt = jnp.transpose(x, (1, 2, 0))
    res = pl.pallas_call(
        _body,
        grid=(_GRID,),
        in_specs=[
            pl.BlockSpec((6, 8, _BL), lambda i: (0, 0, i)),
            pl.BlockSpec((6, 8, 128), lambda i: (0, 0, 0)),
        ],
        out_specs=pl.BlockSpec((6, 8, _BL), lambda i: (0, 0, i)),
        out_shape=jax.ShapeDtypeStruct((6, 8, _N), jnp.float32),
        compiler_params=pltpu.CompilerParams(
            dimension_semantics=("parallel",),
        ),
    )(xt, caddT)
    return jnp.transpose(res, (2, 0, 1))


# speedup vs baseline: 1.0063x; 1.0063x over previous
import jax
import jax.numpy as jnp
import numpy as np
from jax.experimental import pallas as pl
from jax.experimental.pallas import tpu as pltpu

_N = 100000
_BL = 50048
_GRID = -(-_N // _BL)

_T = np.arange(0, 336, 1, dtype=np.float32).reshape(7, 6, 8)
_CADD_T = np.zeros((6, 8, 128), np.float32)
_CADD_T[:, :, 0:7] = _T[[0, 2, 1, 3, 4, 5, 6]].transpose(1, 2, 0)


def _body(x_ref, c_ref, o_ref):
    o_ref[...] = x_ref[...]
    @pl.when(pl.program_id(0) == 0)
    def _():
        o_ref[:, :, 0:128] = o_ref[:, :, 0:128] + c_ref[...]


def kernel(x):
    caddT = jnp.asarray(_CADD_T)
    xt = jnp.transpose(x, (1, 2, 0))
    res = pl.pallas_call(
        _body,
        grid=(_GRID,),
        in_specs=[
            pl.BlockSpec((6, 8, _BL), lambda i: (0, 0, i)),
            pl.BlockSpec((6, 8, 128), lambda i: (0, 0, 0)),
        ],
        out_specs=pl.BlockSpec((6, 8, _BL), lambda i: (0, 0, i)),
        out_shape=jax.ShapeDtypeStruct((6, 8, _N), jnp.float32),
        compiler_params=pltpu.CompilerParams(
            dimension_semantics=("arbitrary",),
        ),
    )(xt, caddT)
    return jnp.transpose(res, (2, 0, 1))
